# baseline (speedup 1.0000x reference)
import functools

import jax
import jax.numpy as jnp
from jax import lax
from jax.experimental import pallas as pl
from jax.experimental.pallas import tpu as pltpu
from jax.experimental.pallas import tpu_sc as plsc

N, C = 1024, 50
D_MODEL = 128
B_TOTAL = N * C

SC_CORES = 2
SC_SUBCORES = 16
NW = SC_CORES * SC_SUBCORES
B_PER_W = B_TOTAL // NW
GATHER_CHUNK = 80
N_CHUNKS = B_PER_W // GATHER_CHUNK

BM = 2048
SUB = 4
BSUB = BM // SUB


def _sc_gather(table, idx_flat):
    mesh = plsc.VectorSubcoreMesh(core_axis_name="c", subcore_axis_name="s")

    @functools.partial(
        pl.kernel,
        mesh=mesh,
        out_type=jax.ShapeDtypeStruct((B_TOTAL, D_MODEL), jnp.float32),
        scratch_types=[
            pltpu.VMEM((B_PER_W,), jnp.int32),
            pltpu.VMEM((GATHER_CHUNK, D_MODEL), jnp.float32),
            pltpu.SemaphoreType.DMA,
        ],
    )
    def k(table_hbm, idx_hbm, out_hbm, idx_v, rows_v, sem):
        wid = lax.axis_index("s") * SC_CORES + lax.axis_index("c")
        base = wid * B_PER_W
        pltpu.sync_copy(idx_hbm.at[pl.ds(base, B_PER_W)], idx_v)

        @pl.loop(0, N_CHUNKS)
        def _(c):
            off = c * GATHER_CHUNK
            pltpu.async_copy(
                table_hbm.at[idx_v.at[pl.ds(off, GATHER_CHUNK)]], rows_v, sem
            ).wait()
            pltpu.sync_copy(rows_v, out_hbm.at[pl.ds(base + off, GATHER_CHUNK)])

    return k(table, idx_flat)


def _mm_body(*refs):
    x_refs = refs[:SUB]
    wt_ref, out_ref = refs[SUB], refs[SUB + 1]
    wt = wt_ref[...].astype(jnp.bfloat16)
    for j in range(SUB):
        x = x_refs[j][...].astype(jnp.bfloat16)
        acc = jnp.dot(x, wt, preferred_element_type=jnp.float32)
        out_ref[pl.ds(j * BSUB, BSUB), :] = acc


def _tc_matmul(x, wt):
    m = x.shape[0]
    grid = (m // BM,)
    x_specs = [
        pl.BlockSpec((BSUB, x.shape[1]), lambda i, j=j: (SUB * i + j, 0))
        for j in range(SUB)
    ]
    return pl.pallas_call(
        _mm_body,
        grid=grid,
        in_specs=x_specs + [
            pl.BlockSpec((wt.shape[0], wt.shape[1]), lambda i: (0, 0)),
        ],
        out_specs=pl.BlockSpec((BM, D_MODEL), lambda i: (i, 0)),
        out_shape=jax.ShapeDtypeStruct((m, D_MODEL), jnp.float32),
        compiler_params=pltpu.CompilerParams(
            dimension_semantics=("parallel",),
        ),
    )(*([x] * SUB), wt)


BM_ADD = 3200


def _add_body(a_ref, b_ref, out_ref):
    out_ref[...] = a_ref[...] + b_ref[...]


def _tc_add(a, b):
    m = a.shape[0]
    grid = (m // BM_ADD,)
    spec = pl.BlockSpec((BM_ADD, D_MODEL), lambda i: (i, 0))
    return pl.pallas_call(
        _add_body,
        grid=grid,
        in_specs=[spec, spec],
        out_specs=spec,
        out_shape=jax.ShapeDtypeStruct((m, D_MODEL), jnp.float32),
        compiler_params=pltpu.CompilerParams(
            dimension_semantics=("parallel",),
        ),
    )(a, b)


def kernel(gene_id, gene_value, E_gene_id, W_gene_value):
    idx_flat = gene_id.T.reshape(-1)
    x = gene_value.transpose(1, 0, 2).reshape(B_TOTAL, -1)
    wt = W_gene_value.T
    emb_cat = _sc_gather(E_gene_id, idx_flat)
    mm = _tc_matmul(x, wt)
    out = _tc_add(mm, emb_cat)
    return out.reshape(C, N, D_MODEL).transpose(1, 0, 2)


# speedup vs baseline: 1.0468x; 1.0468x over previous
import functools

import jax
import jax.numpy as jnp
from jax import lax
from jax.experimental import pallas as pl
from jax.experimental.pallas import tpu as pltpu
from jax.experimental.pallas import tpu_sc as plsc

N, C = 1024, 50
D_MODEL = 128
B_TOTAL = N * C

K_CHUNKS = 2
B_CHUNK = B_TOTAL // K_CHUNKS

SC_CORES = 2
SC_SUBCORES = 16
NW = SC_CORES * SC_SUBCORES
B_PER_W = B_CHUNK // NW
GATHER_CHUNK = 80
N_CHUNKS = B_PER_W // GATHER_CHUNK

BM = 1600
SUB = 4
BSUB = BM // SUB
STEPS_PER_CHUNK = B_CHUNK // BM


def _sc_gather(table, idx_chunk):
    mesh = plsc.VectorSubcoreMesh(core_axis_name="c", subcore_axis_name="s")

    @functools.partial(
        pl.kernel,
        mesh=mesh,
        out_type=jax.ShapeDtypeStruct((B_CHUNK, D_MODEL), jnp.float32),
        scratch_types=[
            pltpu.VMEM((B_PER_W,), jnp.int32),
            pltpu.VMEM((GATHER_CHUNK, D_MODEL), jnp.float32),
            pltpu.SemaphoreType.DMA,
        ],
    )
    def k(table_hbm, idx_hbm, out_hbm, idx_v, rows_v, sem):
        wid = lax.axis_index("s") * SC_CORES + lax.axis_index("c")
        base = wid * B_PER_W
        pltpu.sync_copy(idx_hbm.at[pl.ds(base, B_PER_W)], idx_v)

        @pl.loop(0, N_CHUNKS)
        def _(c):
            off = c * GATHER_CHUNK
            pltpu.async_copy(
                table_hbm.at[idx_v.at[pl.ds(off, GATHER_CHUNK)]], rows_v, sem
            ).wait()
            pltpu.sync_copy(rows_v, out_hbm.at[pl.ds(base + off, GATHER_CHUNK)])

    return k(table, idx_chunk)


def _mm_body_first(*refs):
    x_refs = refs[:SUB]
    wt_ref, cat_ref, out_ref = refs[SUB], refs[SUB + 1], refs[SUB + 2]
    _mm_compute(x_refs, wt_ref, cat_ref, out_ref)


def _mm_body_rest(*refs):
    x_refs = refs[:SUB]
    wt_ref, cat_ref, out_ref = refs[SUB], refs[SUB + 1], refs[SUB + 3]
    _mm_compute(x_refs, wt_ref, cat_ref, out_ref)


def _mm_compute(x_refs, wt_ref, cat_ref, out_ref):
    wt = wt_ref[...].astype(jnp.bfloat16)
    for j in range(SUB):
        x = x_refs[j][...].astype(jnp.bfloat16)
        acc = jnp.dot(x, wt, preferred_element_type=jnp.float32)
        out_ref[pl.ds(j * BSUB, BSUB), :] = (
            acc + cat_ref[pl.ds(j * BSUB, BSUB), :]
        )


def _tc_matmul_add_chunk(chunk, x, wt, cat, prev):
    base = chunk * STEPS_PER_CHUNK
    x_specs = [
        pl.BlockSpec(
            (BSUB, x.shape[1]),
            lambda i, j=j: ((base + i) * SUB + j, 0),
        )
        for j in range(SUB)
    ]
    in_specs = x_specs + [
        pl.BlockSpec((wt.shape[0], wt.shape[1]), lambda i: (0, 0)),
        pl.BlockSpec((BM, D_MODEL), lambda i: (i, 0)),
    ]
    operands = [x] * SUB + [wt, cat]
    kwargs = {}
    if prev is None:
        body = _mm_body_first
    else:
        body = _mm_body_rest
        in_specs = in_specs + [pl.BlockSpec(memory_space=pl.ANY)]
        operands = operands + [prev]
        kwargs["input_output_aliases"] = {SUB + 2: 0}
    return pl.pallas_call(
        body,
        grid=(STEPS_PER_CHUNK,),
        in_specs=in_specs,
        out_specs=pl.BlockSpec((BM, D_MODEL), lambda i: (base + i, 0)),
        out_shape=jax.ShapeDtypeStruct((B_TOTAL, D_MODEL), jnp.float32),
        compiler_params=pltpu.CompilerParams(
            dimension_semantics=("arbitrary",),
        ),
        **kwargs,
    )(*operands)


def kernel(gene_id, gene_value, E_gene_id, W_gene_value):
    idx_flat = gene_id.T.reshape(-1)
    x = gene_value.transpose(1, 0, 2).reshape(B_TOTAL, -1)
    wt = W_gene_value.T
    cats = [
        _sc_gather(E_gene_id, idx_flat[i * B_CHUNK:(i + 1) * B_CHUNK])
        for i in range(K_CHUNKS)
    ]
    out = None
    for i in range(K_CHUNKS):
        out = _tc_matmul_add_chunk(i, x, wt, cats[i], out)
    return out.reshape(C, N, D_MODEL).transpose(1, 0, 2)


# speedup vs baseline: 1.0526x; 1.0056x over previous
import functools

import jax
import jax.numpy as jnp
from jax import lax
from jax.experimental import pallas as pl
from jax.experimental.pallas import tpu as pltpu
from jax.experimental.pallas import tpu_sc as plsc

N, C = 1024, 50
D_MODEL = 128
B_TOTAL = N * C

K_CHUNKS = 4
B_CHUNK = B_TOTAL // K_CHUNKS

SC_CORES = 2
SC_SUBCORES = 16
NW = SC_CORES * SC_SUBCORES
B_PER_W = B_CHUNK // NW
GATHER_CHUNK = 80
N_CHUNKS = B_PER_W // GATHER_CHUNK

BM = 1600
SUB = 4
BSUB = BM // SUB
STEPS_PER_CHUNK = B_CHUNK // BM


def _sc_gather(table, idx_chunk):
    mesh = plsc.VectorSubcoreMesh(core_axis_name="c", subcore_axis_name="s")

    @functools.partial(
        pl.kernel,
        mesh=mesh,
        out_type=jax.ShapeDtypeStruct((B_CHUNK, D_MODEL), jnp.float32),
        scratch_types=[
            pltpu.VMEM((B_PER_W,), jnp.int32),
            pltpu.VMEM((GATHER_CHUNK, D_MODEL), jnp.float32),
            pltpu.SemaphoreType.DMA,
        ],
    )
    def k(table_hbm, idx_hbm, out_hbm, idx_v, rows_v, sem):
        wid = lax.axis_index("s") * SC_CORES + lax.axis_index("c")
        base = wid * B_PER_W
        pltpu.sync_copy(idx_hbm.at[pl.ds(base, B_PER_W)], idx_v)

        @pl.loop(0, N_CHUNKS)
        def _(c):
            off = c * GATHER_CHUNK
            pltpu.async_copy(
                table_hbm.at[idx_v.at[pl.ds(off, GATHER_CHUNK)]], rows_v, sem
            ).wait()
            pltpu.sync_copy(rows_v, out_hbm.at[pl.ds(base + off, GATHER_CHUNK)])

    return k(table, idx_chunk)


def _mm_body_first(*refs):
    x_refs = refs[:SUB]
    wt_ref, cat_ref, out_ref = refs[SUB], refs[SUB + 1], refs[SUB + 2]
    _mm_compute(x_refs, wt_ref, cat_ref, out_ref)


def _mm_body_rest(*refs):
    x_refs = refs[:SUB]
    wt_ref, cat_ref, out_ref = refs[SUB], refs[SUB + 1], refs[SUB + 3]
    _mm_compute(x_refs, wt_ref, cat_ref, out_ref)


def _mm_compute(x_refs, wt_ref, cat_ref, out_ref):
    wt = wt_ref[...].astype(jnp.bfloat16)
    for j in range(SUB):
        x = x_refs[j][...].astype(jnp.bfloat16)
        acc = jnp.dot(x, wt, preferred_element_type=jnp.float32)
        out_ref[pl.ds(j * BSUB, BSUB), :] = (
            acc + cat_ref[pl.ds(j * BSUB, BSUB), :]
        )


def _tc_matmul_add_chunk(chunk, x, wt, cat, prev):
    base = chunk * STEPS_PER_CHUNK
    x_specs = [
        pl.BlockSpec(
            (BSUB, x.shape[1]),
            lambda i, j=j: ((base + i) * SUB + j, 0),
        )
        for j in range(SUB)
    ]
    in_specs = x_specs + [
        pl.BlockSpec((wt.shape[0], wt.shape[1]), lambda i: (0, 0)),
        pl.BlockSpec((BM, D_MODEL), lambda i: (i, 0)),
    ]
    operands = [x] * SUB + [wt, cat]
    kwargs = {}
    if prev is None:
        body = _mm_body_first
    else:
        body = _mm_body_rest
        in_specs = in_specs + [pl.BlockSpec(memory_space=pl.ANY)]
        operands = operands + [prev]
        kwargs["input_output_aliases"] = {SUB + 2: 0}
    return pl.pallas_call(
        body,
        grid=(STEPS_PER_CHUNK,),
        in_specs=in_specs,
        out_specs=pl.BlockSpec((BM, D_MODEL), lambda i: (base + i, 0)),
        out_shape=jax.ShapeDtypeStruct((B_TOTAL, D_MODEL), jnp.float32),
        compiler_params=pltpu.CompilerParams(
            dimension_semantics=("arbitrary",),
        ),
        **kwargs,
    )(*operands)


def kernel(gene_id, gene_value, E_gene_id, W_gene_value):
    idx_flat = gene_id.T.reshape(-1)
    x = gene_value.transpose(1, 0, 2).reshape(B_TOTAL, -1)
    wt = W_gene_value.T
    cats = [
        _sc_gather(E_gene_id, idx_flat[i * B_CHUNK:(i + 1) * B_CHUNK])
        for i in range(K_CHUNKS)
    ]
    out = None
    for i in range(K_CHUNKS):
        out = _tc_matmul_add_chunk(i, x, wt, cats[i], out)
    return out.reshape(C, N, D_MODEL).transpose(1, 0, 2)
